# initial kernel scaffold (unmeasured)
import jax
import jax.numpy as jnp
from jax import lax
from jax.experimental import pallas as pl
from jax.experimental.pallas import tpu as pltpu

N_DEV = 8
N_STEPS = 2 * (N_DEV - 1)
N_SPLIT = 4


def kernel(x, w_mat, scale_x, scale_w):
    m, _ = x.shape
    _, n = w_mat.shape
    m_chunk = m // N_DEV
    n_split = n // N_SPLIT

    if x.dtype != jnp.float8_e5m2:
        x = x.astype(jnp.float8_e5m2)
    if w_mat.dtype != jnp.float8_e5m2:
        w_mat = w_mat.astype(jnp.float8_e5m2)
    scale = (scale_x.astype(jnp.float32) * scale_w.astype(jnp.float32)).reshape(1, 1)

    def body(x_ref, w_ref, scale_ref, out_ref,
             send_buf, recv_buf, send_sems, recv_sems, credit_sem, out_sem):
        d = lax.axis_index("i")
        left = lax.rem(d + N_DEV - 1, N_DEV)
        right = lax.rem(d + 1, N_DEV)

        barrier = pltpu.get_barrier_semaphore()
        for nbr in (left, right):
            pl.semaphore_signal(barrier, 1, device_id=(nbr,),
                                device_id_type=pl.DeviceIdType.MESH)
        pl.semaphore_wait(barrier, 2)

        s = scale_ref[0, 0]

        def compute_into_send(c, add_slot=None, apply_scale=False):
            xc = x_ref[pl.ds(c * m_chunk, m_chunk), :]
            for j in range(N_SPLIT):
                cols = pl.ds(j * n_split, n_split)
                part = lax.dot_general(
                    xc, w_ref[:, cols],
                    (((1,), (0,)), ((), ())),
                    preferred_element_type=jnp.float32,
                )
                if add_slot is not None:
                    part = part + recv_buf[add_slot, :, cols]
                if apply_scale:
                    part = part * s
                send_buf[:, cols] = part

        def store_out(src_ref, c):
            cp = pltpu.make_async_copy(
                src_ref, out_ref.at[pl.ds(c * m_chunk, m_chunk), :], out_sem)
            cp.start()
            cp.wait()

        compute_into_send(d)

        for h in range(N_STEPS):
            slot = h % 2
            if h >= 2:
                pl.semaphore_wait(credit_sem, 1)
            src = send_buf if h <= N_DEV - 1 else recv_buf.at[(h - 1) % 2]
            rdma = pltpu.make_async_remote_copy(
                src_ref=src,
                dst_ref=recv_buf.at[slot],
                send_sem=send_sems.at[slot],
                recv_sem=recv_sems.at[slot],
                device_id=(right,),
                device_id_type=pl.DeviceIdType.MESH,
            )
            rdma.start()
            rdma.wait()
            if h < N_DEV - 1:
                c = lax.rem(d - h - 1 + 2 * N_DEV, N_DEV)
                compute_into_send(c, add_slot=slot,
                                  apply_scale=(h == N_DEV - 2))
                if h == N_DEV - 2:
                    store_out(send_buf, lax.rem(d + 1, N_DEV))
                pl.semaphore_signal(credit_sem, 1, device_id=(left,),
                                    device_id_type=pl.DeviceIdType.MESH)
            else:
                if N_DEV <= h <= N_STEPS - 2:
                    pl.semaphore_signal(credit_sem, 1, device_id=(left,),
                                        device_id_type=pl.DeviceIdType.MESH)
                g = h - (N_DEV - 1)
                c = lax.rem(d - g + 2 * N_DEV, N_DEV)
                store_out(recv_buf.at[slot], c)

    return pl.pallas_call(
        body,
        out_shape=jax.ShapeDtypeStruct((m, n), jnp.float32),
        in_specs=[
            pl.BlockSpec(memory_space=pltpu.VMEM),
            pl.BlockSpec(memory_space=pltpu.VMEM),
            pl.BlockSpec(memory_space=pltpu.SMEM),
        ],
        out_specs=pl.BlockSpec(memory_space=pltpu.ANY),
        scratch_shapes=[
            pltpu.VMEM((m_chunk, n), jnp.float32),
            pltpu.VMEM((2, m_chunk, n), jnp.float32),
            pltpu.SemaphoreType.DMA((2,)),
            pltpu.SemaphoreType.DMA((2,)),
            pltpu.SemaphoreType.REGULAR,
            pltpu.SemaphoreType.DMA,
        ],
        compiler_params=pltpu.CompilerParams(collective_id=0),
    )(x, w_mat, scale)


# baseline (device time: 2718822 ns/iter reference)
import jax
import jax.numpy as jnp
from jax import lax
from jax.experimental import pallas as pl
from jax.experimental.pallas import tpu as pltpu

N_DEV = 8
N_STEPS = 2 * (N_DEV - 1)
N_SPLIT = 4


def kernel(x, w_mat, scale_x, scale_w):
    m, _ = x.shape
    _, n = w_mat.shape
    m_chunk = m // N_DEV
    n_split = n // N_SPLIT

    if x.dtype != jnp.float8_e5m2:
        x = x.astype(jnp.float8_e5m2)
    if w_mat.dtype != jnp.float8_e5m2:
        w_mat = w_mat.astype(jnp.float8_e5m2)
    scale = (scale_x.astype(jnp.float32) * scale_w.astype(jnp.float32)).reshape(1, 1)

    def body(x_ref, w_ref, scale_ref, out_ref,
             send_buf, recv_buf, send_sems, recv_sems, credit_sem, out_sem):
        d = lax.axis_index("i")
        left = lax.rem(d + N_DEV - 1, N_DEV)
        right = lax.rem(d + 1, N_DEV)

        barrier = pltpu.get_barrier_semaphore()
        for nbr in (left, right):
            pl.semaphore_signal(barrier, 1, device_id=(nbr,),
                                device_id_type=pl.DeviceIdType.MESH)
        pl.semaphore_wait(barrier, 2)

        s = scale_ref[0, 0]

        def compute_into_send(c, add_slot=None, apply_scale=False):
            xc = x_ref[pl.ds(c * m_chunk, m_chunk), :]
            for j in range(N_SPLIT):
                cols = pl.ds(j * n_split, n_split)
                part = lax.dot_general(
                    xc, w_ref[:, cols],
                    (((1,), (0,)), ((), ())),
                    preferred_element_type=jnp.float32,
                )
                if add_slot is not None:
                    part = part + recv_buf[add_slot, :, cols]
                if apply_scale:
                    part = part * s
                send_buf[:, cols] = part

        def store_out(src_ref, c):
            cp = pltpu.make_async_copy(
                src_ref, out_ref.at[pl.ds(c * m_chunk, m_chunk), :], out_sem)
            cp.start()
            cp.wait()

        compute_into_send(d)

        for h in range(N_STEPS):
            slot = h % 2
            if h >= 2:
                pl.semaphore_wait(credit_sem, 1)
            src = send_buf if h <= N_DEV - 1 else recv_buf.at[(h - 1) % 2]
            rdma = pltpu.make_async_remote_copy(
                src_ref=src,
                dst_ref=recv_buf.at[slot],
                send_sem=send_sems.at[slot],
                recv_sem=recv_sems.at[slot],
                device_id=(right,),
                device_id_type=pl.DeviceIdType.MESH,
            )
            rdma.start()
            rdma.wait()
            if h < N_DEV - 1:
                c = lax.rem(d - h - 1 + 2 * N_DEV, N_DEV)
                compute_into_send(c, add_slot=slot,
                                  apply_scale=(h == N_DEV - 2))
                if h == N_DEV - 2:
                    store_out(send_buf, lax.rem(d + 1, N_DEV))
                pl.semaphore_signal(credit_sem, 1, device_id=(left,),
                                    device_id_type=pl.DeviceIdType.MESH)
            else:
                if N_DEV <= h <= N_STEPS - 2:
                    pl.semaphore_signal(credit_sem, 1, device_id=(left,),
                                        device_id_type=pl.DeviceIdType.MESH)
                g = h - (N_DEV - 1)
                c = lax.rem(d - g + 2 * N_DEV, N_DEV)
                store_out(recv_buf.at[slot], c)

    return pl.pallas_call(
        body,
        out_shape=jax.ShapeDtypeStruct((m, n), jnp.float32),
        in_specs=[
            pl.BlockSpec(memory_space=pltpu.MemorySpace.VMEM),
            pl.BlockSpec(memory_space=pltpu.MemorySpace.VMEM),
            pl.BlockSpec(memory_space=pltpu.MemorySpace.SMEM),
        ],
        out_specs=pl.BlockSpec(memory_space=pl.ANY),
        scratch_shapes=[
            pltpu.MemorySpace.VMEM((m_chunk, n), jnp.float32),
            pltpu.MemorySpace.VMEM((2, m_chunk, n), jnp.float32),
            pltpu.SemaphoreType.DMA((2,)),
            pltpu.SemaphoreType.DMA((2,)),
            pltpu.SemaphoreType.REGULAR,
            pltpu.SemaphoreType.DMA,
        ],
        compiler_params=pltpu.CompilerParams(
            collective_id=0, vmem_limit_bytes=60 * 1024 * 1024),
    )(x, w_mat, scale)


# device time: 799676 ns/iter; 3.3999x vs baseline; 3.3999x over previous
import jax
import jax.numpy as jnp
from jax import lax
from jax.experimental import pallas as pl
from jax.experimental.pallas import tpu as pltpu

N_DEV = 8
N_STEPS = 2 * (N_DEV - 1)
N_SPLIT = 2


def kernel(x, w_mat, scale_x, scale_w):
    m, _ = x.shape
    _, n = w_mat.shape
    mc = m // N_DEV
    nh = n // 2
    ns = nh // N_SPLIT

    if x.dtype != jnp.float8_e5m2:
        x = x.astype(jnp.float8_e5m2)
    if w_mat.dtype != jnp.float8_e5m2:
        w_mat = w_mat.astype(jnp.float8_e5m2)
    scale = (scale_x.astype(jnp.float32) * scale_w.astype(jnp.float32)).reshape(1, 1)

    def body(x_ref, w_ref, scale_ref, out_ref,
             sbufs, rbufs, accs, send_sems, recv_sems, credit_sems, out_sems):
        d = lax.axis_index("i")
        right = lax.rem(d + 1, N_DEV)
        left = lax.rem(d + N_DEV - 1, N_DEV)
        nbr_to = (right, left)
        nbr_from = (left, right)
        sign = (1, -1)

        def mod(v):
            return lax.rem(v + 4 * N_DEV, N_DEV)

        barrier = pltpu.get_barrier_semaphore()
        for nbr in (left, right):
            pl.semaphore_signal(barrier, 1, device_id=(nbr,),
                                device_id_type=pl.DeviceIdType.MESH)
        pl.semaphore_wait(barrier, 2)

        s = scale_ref[0, 0]

        def gemm_into_acc(r, c):
            xc = x_ref[pl.ds(c * mc, mc), :]
            for j in range(N_SPLIT):
                accs[r, :, pl.ds(j * ns, ns)] = lax.dot_general(
                    xc, w_ref[:, pl.ds(r * nh + j * ns, ns)],
                    (((1,), (0,)), ((), ())),
                    preferred_element_type=jnp.float32,
                )

        def store_out(r, c):
            cp = pltpu.make_async_copy(
                accs.at[r],
                out_ref.at[pl.ds(c * mc, mc), pl.ds(r * nh, nh)],
                out_sems.at[r])
            cp.start()
            return cp

        for r in (0, 1):
            gemm_into_acc(r, d)
            sbufs[r, :, :] = accs[r, :, :].astype(jnp.bfloat16)

        pending = [None, None]
        for h in range(N_STEPS):
            slot = h % 2
            rdmas = []
            for r in (0, 1):
                if h >= 2:
                    pl.semaphore_wait(credit_sems.at[r], 1)
                src = sbufs.at[r] if h <= N_DEV - 1 else rbufs.at[r, (h - 1) % 2]
                rdma = pltpu.make_async_remote_copy(
                    src_ref=src,
                    dst_ref=rbufs.at[r, slot],
                    send_sem=send_sems.at[r, slot],
                    recv_sem=recv_sems.at[r, slot],
                    device_id=(nbr_to[r],),
                    device_id_type=pl.DeviceIdType.MESH,
                )
                rdma.start()
                rdmas.append(rdma)

            if h < N_DEV - 1:
                for r in (0, 1):
                    gemm_into_acc(r, mod(d - sign[r] * (h + 1)))
                for r in (0, 1):
                    rdmas[r].wait_recv()
                    rdmas[r].wait_send()
                    v = accs[r, :, :] + rbufs[r, slot, :, :].astype(jnp.float32)
                    if h == N_DEV - 2:
                        v = v * s
                        accs[r, :, :] = v
                        pending[r] = store_out(r, mod(d + sign[r]))
                    sbufs[r, :, :] = v.astype(jnp.bfloat16)
                    pl.semaphore_signal(credit_sems.at[r], 1,
                                        device_id=(nbr_from[r],),
                                        device_id_type=pl.DeviceIdType.MESH)
            else:
                g = h - (N_DEV - 1)
                for r in (0, 1):
                    rdmas[r].wait_recv()
                for r in (0, 1):
                    if pending[r] is not None:
                        pending[r].wait()
                    accs[r, :, :] = rbufs[r, slot, :, :].astype(jnp.float32)
                    pending[r] = store_out(r, mod(d - sign[r] * g))
                for r in (0, 1):
                    rdmas[r].wait_send()
                    if N_DEV <= h <= N_STEPS - 2:
                        pl.semaphore_signal(credit_sems.at[r], 1,
                                            device_id=(nbr_from[r],),
                                            device_id_type=pl.DeviceIdType.MESH)

        for r in (0, 1):
            pending[r].wait()

    return pl.pallas_call(
        body,
        out_shape=jax.ShapeDtypeStruct((m, n), jnp.float32),
        in_specs=[
            pl.BlockSpec(memory_space=pltpu.MemorySpace.VMEM),
            pl.BlockSpec(memory_space=pltpu.MemorySpace.VMEM),
            pl.BlockSpec(memory_space=pltpu.MemorySpace.SMEM),
        ],
        out_specs=pl.BlockSpec(memory_space=pl.ANY),
        scratch_shapes=[
            pltpu.MemorySpace.VMEM((2, mc, nh), jnp.bfloat16),
            pltpu.MemorySpace.VMEM((2, 2, mc, nh), jnp.bfloat16),
            pltpu.MemorySpace.VMEM((2, mc, nh), jnp.float32),
            pltpu.SemaphoreType.DMA((2, 2)),
            pltpu.SemaphoreType.DMA((2, 2)),
            pltpu.SemaphoreType.REGULAR((2,)),
            pltpu.SemaphoreType.DMA((2,)),
        ],
        compiler_params=pltpu.CompilerParams(
            collective_id=0, vmem_limit_bytes=60 * 1024 * 1024),
    )(x, w_mat, scale)


# device time: 788245 ns/iter; 3.4492x vs baseline; 1.0145x over previous
import jax
import jax.numpy as jnp
from jax import lax
from jax.experimental import pallas as pl
from jax.experimental.pallas import tpu as pltpu

N_DEV = 8
N_STEPS = 2 * (N_DEV - 1)
SIGNS = (1, 1, -1, -1)
N_RING = len(SIGNS)


def kernel(x, w_mat, scale_x, scale_w):
    m, _ = x.shape
    _, n = w_mat.shape
    mc = m // N_DEV
    nw = n // N_RING

    if x.dtype != jnp.float8_e5m2:
        x = x.astype(jnp.float8_e5m2)
    if w_mat.dtype != jnp.float8_e5m2:
        w_mat = w_mat.astype(jnp.float8_e5m2)
    scale = (scale_x.astype(jnp.float32) * scale_w.astype(jnp.float32)).reshape(1, 1)

    def body(x_ref, w_ref, scale_ref, out_ref,
             sbufs, rbufs, accs, send_sems, recv_sems, credit_sems, out_sems):
        d = lax.axis_index("i")
        right = lax.rem(d + 1, N_DEV)
        left = lax.rem(d + N_DEV - 1, N_DEV)
        nbr_to = tuple(right if s > 0 else left for s in SIGNS)
        nbr_from = tuple(left if s > 0 else right for s in SIGNS)

        def mod(v):
            return lax.rem(v + 4 * N_DEV, N_DEV)

        barrier = pltpu.get_barrier_semaphore()
        for nbr in (left, right):
            pl.semaphore_signal(barrier, 1, device_id=(nbr,),
                                device_id_type=pl.DeviceIdType.MESH)
        pl.semaphore_wait(barrier, 2)

        s = scale_ref[0, 0]
        rings = range(N_RING)

        def gemm_into_acc(r, c):
            xc = x_ref[pl.ds(c * mc, mc), :]
            accs[r, :, :] = lax.dot_general(
                xc, w_ref[:, pl.ds(r * nw, nw)],
                (((1,), (0,)), ((), ())),
                preferred_element_type=jnp.float32,
            )

        def store_out(r, c):
            cp = pltpu.make_async_copy(
                accs.at[r],
                out_ref.at[pl.ds(c * mc, mc), pl.ds(r * nw, nw)],
                out_sems.at[r])
            cp.start()
            return cp

        for r in rings:
            gemm_into_acc(r, d)
            sbufs[r, :, :] = accs[r, :, :].astype(jnp.bfloat16)

        pending = [None] * N_RING
        deferred = []
        for h in range(N_STEPS):
            slot = h % 2
            rdmas = []
            for r in rings:
                if h >= 2:
                    pl.semaphore_wait(credit_sems.at[r], 1)
                src = sbufs.at[r] if h <= N_DEV - 1 else rbufs.at[r, (h - 1) % 2]
                rdma = pltpu.make_async_remote_copy(
                    src_ref=src,
                    dst_ref=rbufs.at[r, slot],
                    send_sem=send_sems.at[r, slot],
                    recv_sem=recv_sems.at[r, slot],
                    device_id=(nbr_to[r],),
                    device_id_type=pl.DeviceIdType.MESH,
                )
                rdma.start()
                rdmas.append(rdma)

            for (r, dslot, c) in deferred:
                if pending[r] is not None:
                    pending[r].wait()
                accs[r, :, :] = rbufs[r, dslot, :, :].astype(jnp.float32)
                pending[r] = store_out(r, c)
            flushed = deferred
            deferred = []

            if h < N_DEV - 1:
                for r in rings:
                    gemm_into_acc(r, mod(d - SIGNS[r] * (h + 1)))
                for r in rings:
                    rdmas[r].wait_recv()
                    rdmas[r].wait_send()
                    v = accs[r, :, :] + rbufs[r, slot, :, :].astype(jnp.float32)
                    if h == N_DEV - 2:
                        v = v * s
                        accs[r, :, :] = v
                        pending[r] = store_out(r, mod(d + SIGNS[r]))
                    sbufs[r, :, :] = v.astype(jnp.bfloat16)
                    pl.semaphore_signal(credit_sems.at[r], 1,
                                        device_id=(nbr_from[r],),
                                        device_id_type=pl.DeviceIdType.MESH)
            else:
                g = h - (N_DEV - 1)
                for r in rings:
                    rdmas[r].wait_recv()
                    deferred.append((r, slot, mod(d - SIGNS[r] * g)))
                for r in rings:
                    rdmas[r].wait_send()
                    if N_DEV <= h <= N_STEPS - 2 and flushed:
                        pl.semaphore_signal(credit_sems.at[r], 1,
                                            device_id=(nbr_from[r],),
                                            device_id_type=pl.DeviceIdType.MESH)

        for (r, dslot, c) in deferred:
            if pending[r] is not None:
                pending[r].wait()
            accs[r, :, :] = rbufs[r, dslot, :, :].astype(jnp.float32)
            pending[r] = store_out(r, c)
        for r in rings:
            pending[r].wait()

    return pl.pallas_call(
        body,
        out_shape=jax.ShapeDtypeStruct((m, n), jnp.float32),
        in_specs=[
            pl.BlockSpec(memory_space=pltpu.MemorySpace.VMEM),
            pl.BlockSpec(memory_space=pltpu.MemorySpace.VMEM),
            pl.BlockSpec(memory_space=pltpu.MemorySpace.SMEM),
        ],
        out_specs=pl.BlockSpec(memory_space=pl.ANY),
        scratch_shapes=[
            pltpu.MemorySpace.VMEM((N_RING, mc, nw), jnp.bfloat16),
            pltpu.MemorySpace.VMEM((N_RING, 2, mc, nw), jnp.bfloat16),
            pltpu.MemorySpace.VMEM((N_RING, mc, nw), jnp.float32),
            pltpu.SemaphoreType.DMA((N_RING, 2)),
            pltpu.SemaphoreType.DMA((N_RING, 2)),
            pltpu.SemaphoreType.REGULAR((N_RING,)),
            pltpu.SemaphoreType.DMA((N_RING,)),
        ],
        compiler_params=pltpu.CompilerParams(
            collective_id=0, vmem_limit_bytes=60 * 1024 * 1024),
    )(x, w_mat, scale)


# device time: 756977 ns/iter; 3.5917x vs baseline; 1.0413x over previous
import jax
import jax.numpy as jnp
from jax import lax
from jax.experimental import pallas as pl
from jax.experimental.pallas import tpu as pltpu

N_DEV = 8
N_STEPS = 2 * (N_DEV - 1)
SIGNS = (1, 1, -1, -1)
N_RING = len(SIGNS)


def kernel(x, w_mat, scale_x, scale_w):
    m, _ = x.shape
    _, n = w_mat.shape
    mc = m // N_DEV
    nw = n // N_RING

    if x.dtype != jnp.float8_e5m2:
        x = x.astype(jnp.float8_e5m2)
    if w_mat.dtype != jnp.float8_e5m2:
        w_mat = w_mat.astype(jnp.float8_e5m2)
    scale = (scale_x.astype(jnp.float32) * scale_w.astype(jnp.float32)).reshape(1, 1)

    def body(x_ref, w_ref, scale_ref, out_ref,
             sbufs, rbufs, accs, send_sems, recv_sems, credit_sems, out_sems):
        d = lax.axis_index("i")
        right = lax.rem(d + 1, N_DEV)
        left = lax.rem(d + N_DEV - 1, N_DEV)
        nbr_to = tuple(right if s > 0 else left for s in SIGNS)
        nbr_from = tuple(left if s > 0 else right for s in SIGNS)

        def mod(v):
            return lax.rem(v + 4 * N_DEV, N_DEV)

        barrier = pltpu.get_barrier_semaphore()
        for nbr in (left, right):
            pl.semaphore_signal(barrier, 1, device_id=(nbr,),
                                device_id_type=pl.DeviceIdType.MESH)
        pl.semaphore_wait(barrier, 2)

        s = scale_ref[0, 0]
        rings = range(N_RING)

        def gemm_into_acc(r, c):
            xc = x_ref[pl.ds(c * mc, mc), :]
            accs[r, :, :] = lax.dot_general(
                xc, w_ref[:, pl.ds(r * nw, nw)],
                (((1,), (0,)), ((), ())),
                preferred_element_type=jnp.float32,
            )

        def store_out(r, c):
            cp = pltpu.make_async_copy(
                accs.at[r],
                out_ref.at[pl.ds(c * mc, mc), pl.ds(r * nw, nw)],
                out_sems.at[r])
            cp.start()
            return cp

        def start_send(r, h):
            src = sbufs.at[r] if h <= N_DEV - 1 else rbufs.at[r, (h - 1) % 2]
            rdma = pltpu.make_async_remote_copy(
                src_ref=src,
                dst_ref=rbufs.at[r, h % 2],
                send_sem=send_sems.at[r, h % 2],
                recv_sem=recv_sems.at[r, h % 2],
                device_id=(nbr_to[r],),
                device_id_type=pl.DeviceIdType.MESH,
            )
            rdma.start()
            return rdma

        for r in rings:
            gemm_into_acc(r, d)
            sbufs[r, :, :] = accs[r, :, :].astype(jnp.bfloat16)
        rdmas = [start_send(r, 0) for r in rings]
        for r in rings:
            gemm_into_acc(r, mod(d - SIGNS[r]))

        pending = [None] * N_RING
        deferred = []
        for h in range(N_STEPS):
            slot = h % 2
            for (r, dslot, c) in deferred:
                if pending[r] is not None:
                    pending[r].wait()
                accs[r, :, :] = rbufs[r, dslot, :, :].astype(jnp.float32)
                pending[r] = store_out(r, c)
            deferred = []

            for r in rings:
                rdmas[r].wait_recv()
                if h < N_DEV - 1:
                    rdmas[r].wait_send()
                    v = accs[r, :, :] + rbufs[r, slot, :, :].astype(jnp.float32)
                    if h == N_DEV - 2:
                        v = v * s
                        accs[r, :, :] = v
                        pending[r] = store_out(r, mod(d + SIGNS[r]))
                    sbufs[r, :, :] = v.astype(jnp.bfloat16)
                    pl.semaphore_signal(credit_sems.at[r], 1,
                                        device_id=(nbr_from[r],),
                                        device_id_type=pl.DeviceIdType.MESH)
                    if h + 1 >= 2:
                        pl.semaphore_wait(credit_sems.at[r], 1)
                    rdmas[r] = start_send(r, h + 1)
                    if h < N_DEV - 2:
                        gemm_into_acc(r, mod(d - SIGNS[r] * (h + 2)))
                else:
                    g = h - (N_DEV - 1)
                    deferred.append((r, slot, mod(d - SIGNS[r] * g)))
                    rdmas[r].wait_send()
                    if N_DEV <= h <= N_STEPS - 2:
                        pl.semaphore_signal(credit_sems.at[r], 1,
                                            device_id=(nbr_from[r],),
                                            device_id_type=pl.DeviceIdType.MESH)
                    if h + 1 < N_STEPS:
                        pl.semaphore_wait(credit_sems.at[r], 1)
                        rdmas[r] = start_send(r, h + 1)

        for (r, dslot, c) in deferred:
            if pending[r] is not None:
                pending[r].wait()
            accs[r, :, :] = rbufs[r, dslot, :, :].astype(jnp.float32)
            pending[r] = store_out(r, c)
        for r in rings:
            pending[r].wait()

    return pl.pallas_call(
        body,
        out_shape=jax.ShapeDtypeStruct((m, n), jnp.float32),
        in_specs=[
            pl.BlockSpec(memory_space=pltpu.MemorySpace.VMEM),
            pl.BlockSpec(memory_space=pltpu.MemorySpace.VMEM),
            pl.BlockSpec(memory_space=pltpu.MemorySpace.SMEM),
        ],
        out_specs=pl.BlockSpec(memory_space=pl.ANY),
        scratch_shapes=[
            pltpu.MemorySpace.VMEM((N_RING, mc, nw), jnp.bfloat16),
            pltpu.MemorySpace.VMEM((N_RING, 2, mc, nw), jnp.bfloat16),
            pltpu.MemorySpace.VMEM((N_RING, mc, nw), jnp.float32),
            pltpu.SemaphoreType.DMA((N_RING, 2)),
            pltpu.SemaphoreType.DMA((N_RING, 2)),
            pltpu.SemaphoreType.REGULAR((N_RING,)),
            pltpu.SemaphoreType.DMA((N_RING,)),
        ],
        compiler_params=pltpu.CompilerParams(
            collective_id=0, vmem_limit_bytes=60 * 1024 * 1024),
    )(x, w_mat, scale)


# device time: 751329 ns/iter; 3.6187x vs baseline; 1.0075x over previous
import jax
import jax.numpy as jnp
from jax import lax
from jax.experimental import pallas as pl
from jax.experimental.pallas import tpu as pltpu

N_DEV = 8
N_STEPS = 2 * (N_DEV - 1)
SIGNS = (1, 1, -1, -1)
N_RING = len(SIGNS)


def kernel(x, w_mat, scale_x, scale_w):
    m, _ = x.shape
    _, n = w_mat.shape
    mc = m // N_DEV
    nw = n // N_RING

    if x.dtype != jnp.float8_e5m2:
        x = x.astype(jnp.float8_e5m2)
    if w_mat.dtype != jnp.float8_e5m2:
        w_mat = w_mat.astype(jnp.float8_e5m2)
    scale = (scale_x.astype(jnp.float32) * scale_w.astype(jnp.float32)).reshape(1, 1)

    def body(x_ref, w_ref, scale_ref, out_ref,
             sbufs, rbufs, accs, send_sems, recv_sems, credit_sems, out_sems):
        d = lax.axis_index("i")
        right = lax.rem(d + 1, N_DEV)
        left = lax.rem(d + N_DEV - 1, N_DEV)
        nbr_to = tuple(right if s > 0 else left for s in SIGNS)
        nbr_from = tuple(left if s > 0 else right for s in SIGNS)

        def mod(v):
            return lax.rem(v + 4 * N_DEV, N_DEV)

        barrier = pltpu.get_barrier_semaphore()
        for nbr in (left, right):
            pl.semaphore_signal(barrier, 1, device_id=(nbr,),
                                device_id_type=pl.DeviceIdType.MESH)
        pl.semaphore_wait(barrier, 2)

        s = scale_ref[0, 0]
        rings = (0, 2, 1, 3)

        def gemm_into_acc(r, c):
            xc = x_ref[pl.ds(c * mc, mc), :]
            accs[r, :, :] = lax.dot_general(
                xc, w_ref[:, pl.ds(r * nw, nw)],
                (((1,), (0,)), ((), ())),
                preferred_element_type=jnp.float32,
            )

        def store_out(r, c):
            cp = pltpu.make_async_copy(
                accs.at[r],
                out_ref.at[pl.ds(c * mc, mc), pl.ds(r * nw, nw)],
                out_sems.at[r])
            cp.start()
            return cp

        def start_send(r, h):
            src = sbufs.at[r] if h <= N_DEV - 1 else rbufs.at[r, (h - 1) % 2]
            rdma = pltpu.make_async_remote_copy(
                src_ref=src,
                dst_ref=rbufs.at[r, h % 2],
                send_sem=send_sems.at[r, h % 2],
                recv_sem=recv_sems.at[r, h % 2],
                device_id=(nbr_to[r],),
                device_id_type=pl.DeviceIdType.MESH,
            )
            rdma.start()
            return rdma

        for r in rings:
            gemm_into_acc(r, d)
            sbufs[r, :, :] = accs[r, :, :].astype(jnp.bfloat16)
        rdmas = [start_send(r, 0) for r in rings]
        for r in rings:
            gemm_into_acc(r, mod(d - SIGNS[r]))

        pending = [None] * N_RING
        deferred = []
        for h in range(N_STEPS):
            slot = h % 2
            for (r, dslot, c) in deferred:
                if pending[r] is not None:
                    pending[r].wait()
                accs[r, :, :] = rbufs[r, dslot, :, :].astype(jnp.float32)
                pending[r] = store_out(r, c)
            deferred = []

            for r in rings:
                rdmas[r].wait_recv()
                if h < N_DEV - 1:
                    rdmas[r].wait_send()
                    v = accs[r, :, :] + rbufs[r, slot, :, :].astype(jnp.float32)
                    if h == N_DEV - 2:
                        v = v * s
                        accs[r, :, :] = v
                        pending[r] = store_out(r, mod(d + SIGNS[r]))
                    sbufs[r, :, :] = v.astype(jnp.bfloat16)
                    pl.semaphore_signal(credit_sems.at[r], 1,
                                        device_id=(nbr_from[r],),
                                        device_id_type=pl.DeviceIdType.MESH)
                    if h + 1 >= 2:
                        pl.semaphore_wait(credit_sems.at[r], 1)
                    rdmas[r] = start_send(r, h + 1)
                    if h < N_DEV - 2:
                        gemm_into_acc(r, mod(d - SIGNS[r] * (h + 2)))
                else:
                    g = h - (N_DEV - 1)
                    deferred.append((r, slot, mod(d - SIGNS[r] * g)))
                    rdmas[r].wait_send()
                    if N_DEV <= h <= N_STEPS - 2:
                        pl.semaphore_signal(credit_sems.at[r], 1,
                                            device_id=(nbr_from[r],),
                                            device_id_type=pl.DeviceIdType.MESH)
                    if h + 1 < N_STEPS:
                        pl.semaphore_wait(credit_sems.at[r], 1)
                        rdmas[r] = start_send(r, h + 1)

        for (r, dslot, c) in deferred:
            if pending[r] is not None:
                pending[r].wait()
            accs[r, :, :] = rbufs[r, dslot, :, :].astype(jnp.float32)
            pending[r] = store_out(r, c)
        for r in rings:
            pending[r].wait()

    return pl.pallas_call(
        body,
        out_shape=jax.ShapeDtypeStruct((m, n), jnp.float32),
        in_specs=[
            pl.BlockSpec(memory_space=pltpu.MemorySpace.VMEM),
            pl.BlockSpec(memory_space=pltpu.MemorySpace.VMEM),
            pl.BlockSpec(memory_space=pltpu.MemorySpace.SMEM),
        ],
        out_specs=pl.BlockSpec(memory_space=pl.ANY),
        scratch_shapes=[
            pltpu.MemorySpace.VMEM((N_RING, mc, nw), jnp.bfloat16),
            pltpu.MemorySpace.VMEM((N_RING, 2, mc, nw), jnp.bfloat16),
            pltpu.MemorySpace.VMEM((N_RING, mc, nw), jnp.float32),
            pltpu.SemaphoreType.DMA((N_RING, 2)),
            pltpu.SemaphoreType.DMA((N_RING, 2)),
            pltpu.SemaphoreType.REGULAR((N_RING,)),
            pltpu.SemaphoreType.DMA((N_RING,)),
        ],
        compiler_params=pltpu.CompilerParams(
            collective_id=0, vmem_limit_bytes=60 * 1024 * 1024),
    )(x, w_mat, scale)


# device time: 749810 ns/iter; 3.6260x vs baseline; 1.0020x over previous
import jax
import jax.numpy as jnp
from jax import lax
from jax.experimental import pallas as pl
from jax.experimental.pallas import tpu as pltpu

N_DEV = 8
N_STEPS = 2 * (N_DEV - 1)
SIGNS = (1, 1, 1, 1, -1, -1, -1, -1)
N_RING = len(SIGNS)


def kernel(x, w_mat, scale_x, scale_w):
    m, _ = x.shape
    _, n = w_mat.shape
    mc = m // N_DEV
    nw = n // N_RING

    if x.dtype != jnp.float8_e5m2:
        x = x.astype(jnp.float8_e5m2)
    if w_mat.dtype != jnp.float8_e5m2:
        w_mat = w_mat.astype(jnp.float8_e5m2)
    scale = (scale_x.astype(jnp.float32) * scale_w.astype(jnp.float32)).reshape(1, 1)

    def body(x_ref, w_ref, scale_ref, out_ref,
             sbufs, rbufs, accs, send_sems, recv_sems, credit_sems, out_sems):
        d = lax.axis_index("i")
        right = lax.rem(d + 1, N_DEV)
        left = lax.rem(d + N_DEV - 1, N_DEV)
        nbr_to = tuple(right if s > 0 else left for s in SIGNS)
        nbr_from = tuple(left if s > 0 else right for s in SIGNS)

        def mod(v):
            return lax.rem(v + 4 * N_DEV, N_DEV)

        barrier = pltpu.get_barrier_semaphore()
        for nbr in (left, right):
            pl.semaphore_signal(barrier, 1, device_id=(nbr,),
                                device_id_type=pl.DeviceIdType.MESH)
        pl.semaphore_wait(barrier, 2)

        s = scale_ref[0, 0]
        rings = (0, 4, 1, 5, 2, 6, 3, 7)

        def gemm_into_acc(r, c):
            xc = x_ref[pl.ds(c * mc, mc), :]
            accs[r, :, :] = lax.dot_general(
                xc, w_ref[:, pl.ds(r * nw, nw)],
                (((1,), (0,)), ((), ())),
                preferred_element_type=jnp.float32,
            )

        def store_out(r, c):
            cp = pltpu.make_async_copy(
                accs.at[r],
                out_ref.at[pl.ds(c * mc, mc), pl.ds(r * nw, nw)],
                out_sems.at[r])
            cp.start()
            return cp

        def start_send(r, h):
            src = sbufs.at[r] if h <= N_DEV - 1 else rbufs.at[r, (h - 1) % 2]
            rdma = pltpu.make_async_remote_copy(
                src_ref=src,
                dst_ref=rbufs.at[r, h % 2],
                send_sem=send_sems.at[r, h % 2],
                recv_sem=recv_sems.at[r, h % 2],
                device_id=(nbr_to[r],),
                device_id_type=pl.DeviceIdType.MESH,
            )
            rdma.start()
            return rdma

        for r in rings:
            gemm_into_acc(r, d)
            sbufs[r, :, :] = accs[r, :, :].astype(jnp.bfloat16)
        rdmas = [start_send(r, 0) for r in rings]
        for r in rings:
            gemm_into_acc(r, mod(d - SIGNS[r]))

        pending = [None] * N_RING
        deferred = []
        for h in range(N_STEPS):
            slot = h % 2
            for (r, dslot, c) in deferred:
                if pending[r] is not None:
                    pending[r].wait()
                accs[r, :, :] = rbufs[r, dslot, :, :].astype(jnp.float32)
                pending[r] = store_out(r, c)
            deferred = []

            for r in rings:
                rdmas[r].wait_recv()
                if h < N_DEV - 1:
                    rdmas[r].wait_send()
                    v = accs[r, :, :] + rbufs[r, slot, :, :].astype(jnp.float32)
                    if h == N_DEV - 2:
                        v = v * s
                        accs[r, :, :] = v
                        pending[r] = store_out(r, mod(d + SIGNS[r]))
                    sbufs[r, :, :] = v.astype(jnp.bfloat16)
                    pl.semaphore_signal(credit_sems.at[r], 1,
                                        device_id=(nbr_from[r],),
                                        device_id_type=pl.DeviceIdType.MESH)
                    if h + 1 >= 2:
                        pl.semaphore_wait(credit_sems.at[r], 1)
                    rdmas[r] = start_send(r, h + 1)
                    if h < N_DEV - 2:
                        gemm_into_acc(r, mod(d - SIGNS[r] * (h + 2)))
                else:
                    g = h - (N_DEV - 1)
                    deferred.append((r, slot, mod(d - SIGNS[r] * g)))
                    rdmas[r].wait_send()
                    if N_DEV <= h <= N_STEPS - 2:
                        pl.semaphore_signal(credit_sems.at[r], 1,
                                            device_id=(nbr_from[r],),
                                            device_id_type=pl.DeviceIdType.MESH)
                    if h + 1 < N_STEPS:
                        pl.semaphore_wait(credit_sems.at[r], 1)
                        rdmas[r] = start_send(r, h + 1)

        for (r, dslot, c) in deferred:
            if pending[r] is not None:
                pending[r].wait()
            accs[r, :, :] = rbufs[r, dslot, :, :].astype(jnp.float32)
            pending[r] = store_out(r, c)
        for r in rings:
            pending[r].wait()

    return pl.pallas_call(
        body,
        out_shape=jax.ShapeDtypeStruct((m, n), jnp.float32),
        in_specs=[
            pl.BlockSpec(memory_space=pltpu.MemorySpace.VMEM),
            pl.BlockSpec(memory_space=pltpu.MemorySpace.VMEM),
            pl.BlockSpec(memory_space=pltpu.MemorySpace.SMEM),
        ],
        out_specs=pl.BlockSpec(memory_space=pl.ANY),
        scratch_shapes=[
            pltpu.MemorySpace.VMEM((N_RING, mc, nw), jnp.bfloat16),
            pltpu.MemorySpace.VMEM((N_RING, 2, mc, nw), jnp.bfloat16),
            pltpu.MemorySpace.VMEM((N_RING, mc, nw), jnp.float32),
            pltpu.SemaphoreType.DMA((N_RING, 2)),
            pltpu.SemaphoreType.DMA((N_RING, 2)),
            pltpu.SemaphoreType.REGULAR((N_RING,)),
            pltpu.SemaphoreType.DMA((N_RING,)),
        ],
        compiler_params=pltpu.CompilerParams(
            collective_id=0, vmem_limit_bytes=60 * 1024 * 1024),
    )(x, w_mat, scale)


# device time: 702418 ns/iter; 3.8707x vs baseline; 1.0675x over previous
import jax
import jax.numpy as jnp
from jax import lax
from jax.experimental import pallas as pl
from jax.experimental.pallas import tpu as pltpu

N_DEV = 8
N_STEPS = 2 * (N_DEV - 1)
SIGNS = (1, 1, 1, 1, -1, -1, -1, -1)
N_RING = len(SIGNS)


def kernel(x, w_mat, scale_x, scale_w):
    m, _ = x.shape
    _, n = w_mat.shape
    mc = m // N_DEV
    nw = n // N_RING

    if x.dtype != jnp.float8_e5m2:
        x = x.astype(jnp.float8_e5m2)
    if w_mat.dtype != jnp.float8_e5m2:
        w_mat = w_mat.astype(jnp.float8_e5m2)
    scale = (scale_x.astype(jnp.float32) * scale_w.astype(jnp.float32)).reshape(1, 1)

    def body(x_ref, w_ref, scale_ref, out_ref,
             sbufs, rbufs, accs, send_sems, recv_sems, credit_sems, out_sems):
        d = lax.axis_index("i")
        right = lax.rem(d + 1, N_DEV)
        left = lax.rem(d + N_DEV - 1, N_DEV)
        nbr_to = tuple(right if s > 0 else left for s in SIGNS)
        nbr_from = tuple(left if s > 0 else right for s in SIGNS)

        def mod(v):
            return lax.rem(v + 4 * N_DEV, N_DEV)

        barrier = pltpu.get_barrier_semaphore()
        for nbr in (left, right):
            pl.semaphore_signal(barrier, 1, device_id=(nbr,),
                                device_id_type=pl.DeviceIdType.MESH)
        pl.semaphore_wait(barrier, 2)

        s = scale_ref[0, 0]
        rings = (0, 4, 1, 5, 2, 6, 3, 7)

        def gemm_into_acc(r, c):
            xc = x_ref[pl.ds(c * mc, mc), :]
            accs[r, :, :] = lax.dot_general(
                xc, w_ref[:, pl.ds(r * nw, nw)],
                (((1,), (0,)), ((), ())),
                preferred_element_type=jnp.float32,
            )

        def store_out(r, c, src):
            cp = pltpu.make_async_copy(
                src,
                out_ref.at[pl.ds(c * mc, mc), pl.ds(r * nw, nw)],
                out_sems.at[r])
            cp.start()
            return cp

        def start_send(r, h):
            src = sbufs.at[r] if h <= N_DEV - 1 else rbufs.at[r, (h - 1) % 2]
            rdma = pltpu.make_async_remote_copy(
                src_ref=src,
                dst_ref=rbufs.at[r, h % 2],
                send_sem=send_sems.at[r, h % 2],
                recv_sem=recv_sems.at[r, h % 2],
                device_id=(nbr_to[r],),
                device_id_type=pl.DeviceIdType.MESH,
            )
            rdma.start()
            return rdma

        for r in rings:
            gemm_into_acc(r, d)
            sbufs[r, :, :] = accs[r, :, :].astype(jnp.bfloat16)
        rdmas = [start_send(r, 0) for r in rings]
        for r in rings:
            gemm_into_acc(r, mod(d - SIGNS[r]))

        pending = [None] * N_RING
        deferred = []
        for h in range(N_STEPS):
            slot = h % 2
            for (r, dslot, c) in deferred:
                if pending[r] is not None:
                    pending[r].wait()
                pending[r] = store_out(r, c, rbufs.at[r, dslot])
            deferred = []

            for r in rings:
                rdmas[r].wait_recv()
                if h < N_DEV - 1:
                    rdmas[r].wait_send()
                    v = accs[r, :, :] + rbufs[r, slot, :, :].astype(jnp.float32)
                    if h == N_DEV - 2:
                        v = v * s
                    sbufs[r, :, :] = v.astype(jnp.bfloat16)
                    if h == N_DEV - 2:
                        pending[r] = store_out(r, mod(d + SIGNS[r]),
                                               sbufs.at[r])
                    pl.semaphore_signal(credit_sems.at[r], 1,
                                        device_id=(nbr_from[r],),
                                        device_id_type=pl.DeviceIdType.MESH)
                    if h + 1 >= 2:
                        pl.semaphore_wait(credit_sems.at[r], 1)
                    rdmas[r] = start_send(r, h + 1)
                    if h < N_DEV - 2:
                        gemm_into_acc(r, mod(d - SIGNS[r] * (h + 2)))
                else:
                    g = h - (N_DEV - 1)
                    deferred.append((r, slot, mod(d - SIGNS[r] * g)))
                    rdmas[r].wait_send()
                    if N_DEV <= h <= N_STEPS - 2:
                        if pending[r] is not None:
                            pending[r].wait()
                            pending[r] = None
                        pl.semaphore_signal(credit_sems.at[r], 1,
                                            device_id=(nbr_from[r],),
                                            device_id_type=pl.DeviceIdType.MESH)
                    if h + 1 < N_STEPS:
                        pl.semaphore_wait(credit_sems.at[r], 1)
                        rdmas[r] = start_send(r, h + 1)

        for (r, dslot, c) in deferred:
            if pending[r] is not None:
                pending[r].wait()
            pending[r] = store_out(r, c, rbufs.at[r, dslot])
        for r in rings:
            if pending[r] is not None:
                pending[r].wait()

    return pl.pallas_call(
        body,
        out_shape=jax.ShapeDtypeStruct((m, n), jnp.bfloat16),
        in_specs=[
            pl.BlockSpec(memory_space=pltpu.MemorySpace.VMEM),
            pl.BlockSpec(memory_space=pltpu.MemorySpace.VMEM),
            pl.BlockSpec(memory_space=pltpu.MemorySpace.SMEM),
        ],
        out_specs=pl.BlockSpec(memory_space=pl.ANY),
        scratch_shapes=[
            pltpu.MemorySpace.VMEM((N_RING, mc, nw), jnp.bfloat16),
            pltpu.MemorySpace.VMEM((N_RING, 2, mc, nw), jnp.bfloat16),
            pltpu.MemorySpace.VMEM((N_RING, mc, nw), jnp.float32),
            pltpu.SemaphoreType.DMA((N_RING, 2)),
            pltpu.SemaphoreType.DMA((N_RING, 2)),
            pltpu.SemaphoreType.REGULAR((N_RING,)),
            pltpu.SemaphoreType.DMA((N_RING,)),
        ],
        compiler_params=pltpu.CompilerParams(
            collective_id=0, vmem_limit_bytes=60 * 1024 * 1024),
    )(x, w_mat, scale)
